# baseline (speedup 1.0000x reference)
import functools

import jax
import jax.numpy as jnp
from jax.experimental import pallas as pl
from jax.experimental.pallas import tpu as pltpu

N_NODES = 10000
N_EDGES = 2000
D = 128
TN = 1000


def _contract0(a, b):
    return jax.lax.dot_general(
        a, b, dimension_numbers=(((0,), (0,)), ((), ())),
        preferred_element_type=jnp.float32)


def _pass1_kernel(b_ref, x0_ref, w1_ref, h1_ref, acc_ref):
    i = pl.program_id(0)

    @pl.when(i == 0)
    def _():
        acc_ref[...] = jnp.zeros_like(acc_ref)

    acc_ref[...] += _contract0(b_ref[...], x0_ref[...])

    @pl.when(i == pl.num_programs(0) - 1)
    def _():
        h1_ref[...] = jnp.dot(acc_ref[...], w1_ref[...],
                              preferred_element_type=jnp.float32)


def _pass2_kernel(b_ref, h1_ref, w2_ref, x1_ref, h2_ref, acc_ref):
    i = pl.program_id(0)

    @pl.when(i == 0)
    def _():
        acc_ref[...] = jnp.zeros_like(acc_ref)

    x0b = jnp.dot(b_ref[...], h1_ref[...], preferred_element_type=jnp.float32)
    acc_ref[...] += _contract0(b_ref[...], x0b)

    @pl.when(i == pl.num_programs(0) - 1)
    def _():
        x1_ref[...] = acc_ref[...]
        h2_ref[...] = jnp.dot(acc_ref[...], w2_ref[...],
                              preferred_element_type=jnp.float32)


def _pass3_kernel(b_ref, h2_ref, out_ref):
    out_ref[...] = jnp.dot(b_ref[...], h2_ref[...],
                           preferred_element_type=jnp.float32)


@jax.jit
def kernel(x_0, incidence_1, W1, W2):
    n, e = incidence_1.shape
    d = x_0.shape[1]
    nt = n // TN
    f32 = jnp.float32

    h1 = pl.pallas_call(
        _pass1_kernel,
        grid=(nt,),
        in_specs=[
            pl.BlockSpec((TN, e), lambda i: (i, 0)),
            pl.BlockSpec((TN, d), lambda i: (i, 0)),
            pl.BlockSpec((d, d), lambda i: (0, 0)),
        ],
        out_specs=pl.BlockSpec((e, d), lambda i: (0, 0)),
        out_shape=jax.ShapeDtypeStruct((e, d), f32),
        scratch_shapes=[pltpu.VMEM((e, d), f32)],
        compiler_params=pltpu.CompilerParams(
            dimension_semantics=("arbitrary",)),
    )(incidence_1, x_0, W1)

    x1_out, h2 = pl.pallas_call(
        _pass2_kernel,
        grid=(nt,),
        in_specs=[
            pl.BlockSpec((TN, e), lambda i: (i, 0)),
            pl.BlockSpec((e, d), lambda i: (0, 0)),
            pl.BlockSpec((d, d), lambda i: (0, 0)),
        ],
        out_specs=[
            pl.BlockSpec((e, d), lambda i: (0, 0)),
            pl.BlockSpec((e, d), lambda i: (0, 0)),
        ],
        out_shape=[
            jax.ShapeDtypeStruct((e, d), f32),
            jax.ShapeDtypeStruct((e, d), f32),
        ],
        scratch_shapes=[pltpu.VMEM((e, d), f32)],
        compiler_params=pltpu.CompilerParams(
            dimension_semantics=("arbitrary",)),
    )(incidence_1, h1, W2)

    x0_out = pl.pallas_call(
        _pass3_kernel,
        grid=(nt,),
        in_specs=[
            pl.BlockSpec((TN, e), lambda i: (i, 0)),
            pl.BlockSpec((e, d), lambda i: (0, 0)),
        ],
        out_specs=pl.BlockSpec((TN, d), lambda i: (i, 0)),
        out_shape=jax.ShapeDtypeStruct((n, d), f32),
        compiler_params=pltpu.CompilerParams(
            dimension_semantics=("parallel",)),
    )(incidence_1, h2)

    return (x0_out, x1_out)
